# baseline (speedup 1.0000x reference)
import functools

import jax
import jax.numpy as jnp
from jax import lax
from jax.experimental import pallas as pl
from jax.experimental.pallas import tpu as pltpu
from jax.experimental.pallas import tpu_sc as plsc

B = 1024
V = 100000
L = 50

NC = 2
NS = 16
NW = NC * NS
PER_W = B * L // NW
CHUNK = 64

_CHUNKS = []
_off = 0
while _off < PER_W:
    _c = min(CHUNK, PER_W - _off)
    _CHUNKS.append((_off, _c))
    _off += _c

_H1 = 768
_H2 = PER_W - _H1

_mesh = plsc.VectorSubcoreMesh(core_axis_name="c", subcore_axis_name="s")


@functools.partial(
    pl.kernel,
    out_type=jax.ShapeDtypeStruct((B * L,), jnp.float32),
    mesh=_mesh,
    scratch_types=[
        pltpu.VMEM((PER_W,), jnp.int32),
        pltpu.VMEM((PER_W,), jnp.float32),
        pltpu.SemaphoreType.DMA,
        pltpu.SemaphoreType.DMA,
        pltpu.SemaphoreType.DMA,
    ],
)
def _pg_gather(idx_hbm, pred_hbm, out_hbm, idx_v, val_v, sem_in, sem_g, sem_o):
    wid = lax.axis_index("s") * NC + lax.axis_index("c")
    base = wid * PER_W
    cp1 = pltpu.async_copy(idx_hbm.at[pl.ds(base, _H1)],
                           idx_v.at[pl.ds(0, _H1)], sem_in)
    cp2 = pltpu.async_copy(idx_hbm.at[pl.ds(base + _H1, _H2)],
                           idx_v.at[pl.ds(_H1, _H2)], sem_in)
    cp1.wait()
    g_cps = [
        pltpu.async_copy(pred_hbm.at[idx_v.at[pl.ds(o, c)]],
                         val_v.at[pl.ds(o, c)], sem_g)
        for o, c in _CHUNKS if o < _H1
    ]
    cp2.wait()
    g_cps += [
        pltpu.async_copy(pred_hbm.at[idx_v.at[pl.ds(o, c)]],
                         val_v.at[pl.ds(o, c)], sem_g)
        for o, c in _CHUNKS if o >= _H1
    ]
    for cp in g_cps[:6]:
        cp.wait()
    o1 = pltpu.async_copy(val_v.at[pl.ds(0, _H1)],
                          out_hbm.at[pl.ds(base, _H1)], sem_o)
    for cp in g_cps[6:]:
        cp.wait()
    o2 = pltpu.async_copy(val_v.at[pl.ds(_H1, _H2)],
                          out_hbm.at[pl.ds(base + _H1, _H2)], sem_o)
    o1.wait()
    o2.wait()


def kernel(pred, target, reward):
    t = target.astype(jnp.int32)
    i = jnp.arange(B, dtype=jnp.int32)[:, None]
    n = ((t >> 3) << 13) + ((i >> 7) << 10) + ((t & 7) << 7) + (i & 127)
    pred_lin = pred.reshape(8, 128, V // 8, 8).transpose(2, 0, 3, 1).reshape(-1)
    val = _pg_gather(n.reshape(-1), pred_lin)
    return jnp.sum(val * reward.reshape(-1)) * jnp.float32(-1.0 / B)


# speedup vs baseline: 1.0063x; 1.0063x over previous
import functools

import jax
import jax.numpy as jnp
from jax import lax
from jax.experimental import pallas as pl
from jax.experimental.pallas import tpu as pltpu
from jax.experimental.pallas import tpu_sc as plsc

B = 1024
V = 100000
L = 50

NC = 2
NS = 16
NW = NC * NS
PER_W = B * L // NW
CHUNK = 128

_CHUNKS = []
_off = 0
while _off < PER_W:
    _c = min(CHUNK, PER_W - _off)
    _CHUNKS.append((_off, _c))
    _off += _c

_H1 = 768
_H2 = PER_W - _H1

_mesh = plsc.VectorSubcoreMesh(core_axis_name="c", subcore_axis_name="s")


@functools.partial(
    pl.kernel,
    out_type=jax.ShapeDtypeStruct((B * L,), jnp.float32),
    mesh=_mesh,
    scratch_types=[
        pltpu.VMEM((PER_W,), jnp.int32),
        pltpu.VMEM((PER_W,), jnp.float32),
        pltpu.SemaphoreType.DMA,
        pltpu.SemaphoreType.DMA,
        pltpu.SemaphoreType.DMA,
    ],
)
def _pg_gather(idx_hbm, pred_hbm, out_hbm, idx_v, val_v, sem_in, sem_g, sem_o):
    wid = lax.axis_index("s") * NC + lax.axis_index("c")
    base = wid * PER_W
    cp1 = pltpu.async_copy(idx_hbm.at[pl.ds(base, _H1)],
                           idx_v.at[pl.ds(0, _H1)], sem_in)
    cp2 = pltpu.async_copy(idx_hbm.at[pl.ds(base + _H1, _H2)],
                           idx_v.at[pl.ds(_H1, _H2)], sem_in)
    cp1.wait()
    g_cps = [
        pltpu.async_copy(pred_hbm.at[idx_v.at[pl.ds(o, c)]],
                         val_v.at[pl.ds(o, c)], sem_g)
        for o, c in _CHUNKS if o < _H1
    ]
    cp2.wait()
    g_cps += [
        pltpu.async_copy(pred_hbm.at[idx_v.at[pl.ds(o, c)]],
                         val_v.at[pl.ds(o, c)], sem_g)
        for o, c in _CHUNKS if o >= _H1
    ]
    for cp in g_cps[:6]:
        cp.wait()
    o1 = pltpu.async_copy(val_v.at[pl.ds(0, _H1)],
                          out_hbm.at[pl.ds(base, _H1)], sem_o)
    for cp in g_cps[6:]:
        cp.wait()
    o2 = pltpu.async_copy(val_v.at[pl.ds(_H1, _H2)],
                          out_hbm.at[pl.ds(base + _H1, _H2)], sem_o)
    o1.wait()
    o2.wait()


def kernel(pred, target, reward):
    t = target.astype(jnp.int32)
    i = jnp.arange(B, dtype=jnp.int32)[:, None]
    n = ((t >> 3) << 13) + ((i >> 7) << 10) + ((t & 7) << 7) + (i & 127)
    pred_lin = pred.reshape(8, 128, V // 8, 8).transpose(2, 0, 3, 1).reshape(-1)
    val = _pg_gather(n.reshape(-1), pred_lin)
    return jnp.sum(val * reward.reshape(-1)) * jnp.float32(-1.0 / B)
